# Initial kernel scaffold; baseline (speedup 1.0000x reference)
import functools

import jax
import jax.numpy as jnp
from jax import lax
from jax.experimental import pallas as pl
from jax.experimental.pallas import tpu as pltpu
from jax.experimental.pallas import tpu_sc as plsc

NC = 2
NS = 16
NW = NC * NS
L = 16
CHUNK = 32
GATHER_W = 128


def _sc_body(nchunk, n_neg, dim,
             u_pos_h, v_pos_h, v_neg_h, u_w_h, v_w_h,
             pos_out_h, neg_out_h,
             idx_u, idx_vp, idx_ng, u_rows, vp_rows, ng_rows,
             pos_part, neg_part, sem):
  neg_g = CHUNK * n_neg // GATHER_W
  dvr = dim // L
  cid = lax.axis_index("c")
  sid = lax.axis_index("s")
  wid = sid * NC + cid

  pltpu.sync_copy(u_pos_h.at[wid], idx_u)
  pltpu.sync_copy(v_pos_h.at[wid], idx_vp)
  pltpu.sync_copy(v_neg_h.at[wid], idx_ng)

  def chunk_body(c, carry):
    cp_u = pltpu.async_copy(u_w_h.at[idx_u.at[c]], u_rows, sem)
    cp_vp = pltpu.async_copy(v_w_h.at[idx_vp.at[c]], vp_rows, sem)
    cps = [
        pltpu.async_copy(v_w_h.at[idx_ng.at[c, g]],
                         ng_rows.at[pl.ds(g * GATHER_W, GATHER_W)], sem)
        for g in range(neg_g)
    ]
    cp_u.wait()
    cp_vp.wait()
    for cp in cps:
      cp.wait()

    def elem_body(e, carry2):
      u = [u_rows[e, pl.ds(j * L, L)] for j in range(dvr)]
      p = u[0] * vp_rows[e, pl.ds(0, L)]
      for j in range(1, dvr):
        p = p + u[j] * vp_rows[e, pl.ds(j * L, L)]
      a = [jnp.zeros((L,), jnp.float32) for _ in range(dvr)]
      for n in range(n_neg):
        f = e * n_neg + n
        for j in range(dvr):
          a[j] = a[j] + ng_rows[f, pl.ds(j * L, L)]
      q = u[0] * a[0]
      for j in range(1, dvr):
        q = q + u[j] * a[j]
      row = c * CHUNK + e
      pos_part[row, :] = p
      neg_part[row, :] = q
      return carry2

    lax.fori_loop(0, CHUNK, elem_body, 0)
    return carry

  lax.fori_loop(0, nchunk, chunk_body, 0)

  pltpu.sync_copy(pos_part, pos_out_h.at[wid])
  pltpu.sync_copy(neg_part, neg_out_h.at[wid])


def _tc_body(pos_ref, neg_ref, out_ref):
  i = pl.program_id(0)
  ps = jnp.sum(pos_ref[...], axis=1, keepdims=True)
  ns = jnp.sum(neg_ref[...], axis=1, keepdims=True)
  cost = jax.nn.log_sigmoid(ps) + jax.nn.log_sigmoid(-ns)
  s = jnp.sum(cost)

  @pl.when(i == 0)
  def _():
    out_ref[0, 0] = 0.0

  out_ref[0, 0] += s


@jax.jit
def kernel(u_pos, v_pos, v_neg, batch_size, u_weight, v_weight):
  b = u_pos.shape[0]
  n_neg = v_neg.shape[1]
  dim = u_weight.shape[1]
  bpw = b // NW
  nchunk = bpw // CHUNK

  u_pos_r = u_pos.astype(jnp.int32).reshape(NW, nchunk, CHUNK)
  v_pos_r = v_pos.astype(jnp.int32).reshape(NW, nchunk, CHUNK)
  v_neg_r = v_neg.astype(jnp.int32).reshape(
      NW, nchunk, CHUNK * n_neg // GATHER_W, GATHER_W)

  mesh = plsc.VectorSubcoreMesh(
      core_axis_name="c", subcore_axis_name="s",
      num_cores=NC, num_subcores=NS)
  part_ty = jax.ShapeDtypeStruct((NW, bpw, L), jnp.float32)
  sc = pl.kernel(
      functools.partial(_sc_body, nchunk, n_neg, dim),
      out_type=(part_ty, part_ty),
      mesh=mesh,
      scratch_types=[
          pltpu.VMEM((nchunk, CHUNK), jnp.int32),
          pltpu.VMEM((nchunk, CHUNK), jnp.int32),
          pltpu.VMEM((nchunk, CHUNK * n_neg // GATHER_W, GATHER_W), jnp.int32),
          pltpu.VMEM((CHUNK, dim), jnp.float32),
          pltpu.VMEM((CHUNK, dim), jnp.float32),
          pltpu.VMEM((CHUNK * n_neg, dim), jnp.float32),
          pltpu.VMEM((bpw, L), jnp.float32),
          pltpu.VMEM((bpw, L), jnp.float32),
          pltpu.SemaphoreType.DMA,
      ],
  )
  pos_part, neg_part = sc(u_pos_r, v_pos_r, v_neg_r, u_weight, v_weight)

  rows_blk = 2048
  total = pl.pallas_call(
      _tc_body,
      grid=(b // rows_blk,),
      in_specs=[
          pl.BlockSpec((rows_blk, L), lambda i: (i, 0)),
          pl.BlockSpec((rows_blk, L), lambda i: (i, 0)),
      ],
      out_specs=pl.BlockSpec((1, 1), lambda i: (0, 0)),
      out_shape=jax.ShapeDtypeStruct((1, 1), jnp.float32),
  )(pos_part.reshape(b, L), neg_part.reshape(b, L))

  return -total[0, 0] / batch_size


# speedup vs baseline: 5.2441x; 5.2441x over previous
import functools

import jax
import jax.numpy as jnp
from jax import lax
from jax.experimental import pallas as pl
from jax.experimental.pallas import tpu as pltpu
from jax.experimental.pallas import tpu_sc as plsc

NC = 2
NS = 16
NW = NC * NS
L = 16
CHUNK = 32
GATHER_W = 128


def _sc_body(nchunk, n_neg, dim,
             u_pos_h, v_pos_h, v_neg_h, u_w_h, v_w_h,
             pos_out_h, neg_out_h,
             idx_u, idx_vp, idx_ng, u_rows, vp_rows, ng_rows,
             pos_part, neg_part, sem):
  neg_g = CHUNK * n_neg // GATHER_W
  dvr = dim // L
  cid = lax.axis_index("c")
  sid = lax.axis_index("s")
  wid = sid * NC + cid

  pltpu.sync_copy(u_pos_h.at[wid], idx_u)
  pltpu.sync_copy(v_pos_h.at[wid], idx_vp)
  pltpu.sync_copy(v_neg_h.at[wid], idx_ng)

  def chunk_body(c, carry):
    cp_u = pltpu.async_copy(u_w_h.at[idx_u.at[c]], u_rows, sem)
    cp_vp = pltpu.async_copy(v_w_h.at[idx_vp.at[c]], vp_rows, sem)
    cps = [
        pltpu.async_copy(v_w_h.at[idx_ng.at[c, g]],
                         ng_rows.at[pl.ds(g * GATHER_W, GATHER_W)], sem)
        for g in range(neg_g)
    ]
    cp_u.wait()
    cp_vp.wait()
    for cp in cps:
      cp.wait()

    def elem_body(e, carry2):
      u = [u_rows[e, pl.ds(j * L, L)] for j in range(dvr)]
      p = u[0] * vp_rows[e, pl.ds(0, L)]
      for j in range(1, dvr):
        p = p + u[j] * vp_rows[e, pl.ds(j * L, L)]
      a = [jnp.zeros((L,), jnp.float32) for _ in range(dvr)]
      for n in range(n_neg):
        f = e * n_neg + n
        for j in range(dvr):
          a[j] = a[j] + ng_rows[f, pl.ds(j * L, L)]
      q = u[0] * a[0]
      for j in range(1, dvr):
        q = q + u[j] * a[j]
      row = c * CHUNK + e
      pos_part[row, :] = p
      neg_part[row, :] = q
      return carry2

    lax.fori_loop(0, CHUNK, elem_body, 0)
    return carry

  lax.fori_loop(0, nchunk, chunk_body, 0)

  pltpu.sync_copy(pos_part, pos_out_h.at[wid])
  pltpu.sync_copy(neg_part, neg_out_h.at[wid])


def _tc_body(pos_ref, neg_ref, out_ref):
  i = pl.program_id(0)
  ps = jnp.sum(pos_ref[...], axis=1, keepdims=True)
  ns = jnp.sum(neg_ref[...], axis=1, keepdims=True)
  cost = jax.nn.log_sigmoid(ps) + jax.nn.log_sigmoid(-ns)
  s = jnp.sum(cost).reshape(1, 1)

  @pl.when(i == 0)
  def _():
    out_ref[...] = jnp.zeros((1, 1), jnp.float32)

  out_ref[...] += s


@jax.jit
def kernel(u_pos, v_pos, v_neg, batch_size, u_weight, v_weight):
  b = u_pos.shape[0]
  n_neg = v_neg.shape[1]
  dim = u_weight.shape[1]
  bpw = b // NW
  nchunk = bpw // CHUNK

  u_pos_r = u_pos.astype(jnp.int32).reshape(NW, nchunk, CHUNK)
  v_pos_r = v_pos.astype(jnp.int32).reshape(NW, nchunk, CHUNK)
  v_neg_r = v_neg.astype(jnp.int32).reshape(
      NW, nchunk, CHUNK * n_neg // GATHER_W, GATHER_W)

  mesh = plsc.VectorSubcoreMesh(
      core_axis_name="c", subcore_axis_name="s",
      num_cores=NC, num_subcores=NS)
  part_ty = jax.ShapeDtypeStruct((NW, bpw, L), jnp.float32)
  sc = pl.kernel(
      functools.partial(_sc_body, nchunk, n_neg, dim),
      out_type=(part_ty, part_ty),
      mesh=mesh,
      compiler_params=pltpu.CompilerParams(use_tc_tiling_on_sc=False),
      scratch_types=[
          pltpu.VMEM((nchunk, CHUNK), jnp.int32),
          pltpu.VMEM((nchunk, CHUNK), jnp.int32),
          pltpu.VMEM((nchunk, CHUNK * n_neg // GATHER_W, GATHER_W), jnp.int32),
          pltpu.VMEM((CHUNK, dim), jnp.float32),
          pltpu.VMEM((CHUNK, dim), jnp.float32),
          pltpu.VMEM((CHUNK * n_neg, dim), jnp.float32),
          pltpu.VMEM((bpw, L), jnp.float32),
          pltpu.VMEM((bpw, L), jnp.float32),
          pltpu.SemaphoreType.DMA,
      ],
  )
  pos_part, neg_part = sc(u_pos_r, v_pos_r, v_neg_r, u_weight, v_weight)

  rows_blk = 2048
  total = pl.pallas_call(
      _tc_body,
      grid=(b // rows_blk,),
      in_specs=[
          pl.BlockSpec((rows_blk, L), lambda i: (i, 0)),
          pl.BlockSpec((rows_blk, L), lambda i: (i, 0)),
      ],
      out_specs=pl.BlockSpec((1, 1), lambda i: (0, 0)),
      out_shape=jax.ShapeDtypeStruct((1, 1), jnp.float32),
  )(pos_part.reshape(b, L), neg_part.reshape(b, L))

  return -total[0, 0] / batch_size
